# baseline (speedup 1.0000x reference)
import functools

import jax
import jax.numpy as jnp
from jax import lax
from jax.experimental import pallas as pl
from jax.experimental.pallas import tpu as pltpu
from jax.experimental.pallas import tpu_sc as plsc

MM = 1000000
DD = 64
BB = 16384
NC, NS, LL = 2, 16, 16
NW = NC * NS
RPW = MM // NW
BPW = BB // NW
PT = 31264
NCH = 8

_mesh = plsc.VectorSubcoreMesh(core_axis_name="c", subcore_axis_name="s")
_sc_params = pltpu.CompilerParams(
    use_tc_tiling_on_sc=False, needs_layout_passes=False)


_CROWS = DD * MM // 128
_CBLK = 4000
_copy_body = lambda i_ref, o_ref: o_ref.__setitem__((...,), i_ref[...])
_copy_call = pl.pallas_call(
    _copy_body,
    grid=(_CROWS // _CBLK,),
    in_specs=[pl.BlockSpec((_CBLK, 128), lambda i: (i, 0))],
    out_specs=pl.BlockSpec((_CBLK, 128), lambda i: (i, 0)),
    out_shape=jax.ShapeDtypeStruct((_CROWS, 128), jnp.float32),
)


@functools.partial(
    pl.kernel,
    mesh=_mesh,
    out_type=jax.ShapeDtypeStruct((DD, BB), jnp.float32),
    compiler_params=_sc_params,
    scratch_types=[
        pltpu.VMEM((BPW,), jnp.int32),
        pltpu.VMEM((DD * BPW,), jnp.int32),
        pltpu.VMEM((DD, BPW), jnp.float32),
        pltpu.SemaphoreType.DMA,
    ],
)
def _sc_gather(t1d, ids, out2, idx_v, kbuf, buf, sem):
    wid = lax.axis_index("s") * NC + lax.axis_index("c")
    b0 = wid * BPW
    pltpu.sync_copy(ids.at[pl.ds(b0, BPW)], idx_v)

    def build(i, _):
        k = i >> 5
        c2 = i & 31
        v = idx_v[pl.ds(c2 * LL, LL)] + k * MM
        kbuf[pl.ds(i * LL, LL)] = v
        return 0

    lax.fori_loop(0, DD * 32, build, 0, unroll=8)

    nd = DD * (BPW // 128)

    def fire(d, _):
        pltpu.async_copy(
            t1d.at[kbuf.at[pl.ds(d * 128, 128)]],
            buf.at[d >> 2, pl.ds((d & 3) * 128, 128)],
            sem,
        )
        return 0

    lax.fori_loop(0, nd, fire, 0)

    def drain(d, _):
        pltpu.make_async_copy(
            t1d.at[pl.ds(0, 128)], buf.at[0, pl.ds(0, 128)], sem).wait()
        return 0

    lax.fori_loop(0, nd, drain, 0)
    pltpu.sync_copy(buf, out2.at[:, pl.ds(b0, BPW)])


def _gru_body(h_ref, m_ref, wit_ref, wht_ref, bi_ref, bh_ref, o_ref):
    h = h_ref[...]
    gi = lax.dot_general(wit_ref[...], m_ref[...], (((0,), (0,)), ((), ())),
                         preferred_element_type=jnp.float32) + bi_ref[...]
    gh = lax.dot_general(wht_ref[...], h, (((0,), (0,)), ((), ())),
                         preferred_element_type=jnp.float32) + bh_ref[...]
    r = jax.nn.sigmoid(gi[0:DD, :] + gh[0:DD, :])
    z = jax.nn.sigmoid(gi[DD:2 * DD, :] + gh[DD:2 * DD, :])
    n = jnp.tanh(gi[2 * DD:3 * DD, :] + r * gh[2 * DD:3 * DD, :])
    o_ref[...] = (1.0 - z) * n + z * h


_GBLK = 2048
_gru_call = pl.pallas_call(
    _gru_body,
    grid=(BB // _GBLK,),
    in_specs=[
        pl.BlockSpec((DD, _GBLK), lambda i: (0, i)),
        pl.BlockSpec((DD, _GBLK), lambda i: (0, i)),
        pl.BlockSpec((DD, 3 * DD), lambda i: (0, 0)),
        pl.BlockSpec((DD, 3 * DD), lambda i: (0, 0)),
        pl.BlockSpec((3 * DD, 1), lambda i: (0, 0)),
        pl.BlockSpec((3 * DD, 1), lambda i: (0, 0)),
    ],
    out_specs=pl.BlockSpec((DD, _GBLK), lambda i: (0, i)),
    out_shape=jax.ShapeDtypeStruct((DD, BB), jnp.float32),
)


@functools.partial(
    pl.kernel,
    mesh=_mesh,
    out_type=(),
    compiler_params=_sc_params,
    scratch_types=[
        pltpu.VMEM((BB,), jnp.int32),
        pltpu.VMEM((PT,), jnp.int32),
        pltpu.VMEM((NCH, 128), jnp.int32),
        pltpu.VMEM((NCH, 128), jnp.int32),
        pltpu.VMEM((DD * 128,), jnp.int32),
        pltpu.VMEM((DD, 128), jnp.int32),
        pltpu.VMEM((DD * 128,), jnp.float32),
        pltpu.SemaphoreType.DMA,
        pltpu.SemaphoreType.DMA,
    ],
)
def _sc_scatter(tref, ids, upd1d, ids_v, postab, gidx, sidx, idxg, idxs,
                vals, gsem, ssem):
    wid = lax.axis_index("s") * NC + lax.axis_index("c")
    base = wid * RPW
    zeros = jnp.zeros((LL,), jnp.int32)

    pltpu.sync_copy(ids, ids_v)

    def za(i, _):
        postab[pl.ds(i * LL, LL)] = zeros
        return 0

    lax.fori_loop(0, PT // LL, za, 0, unroll=8)

    def sb(i, _):
        idv = ids_v[pl.ds(i * LL, LL)]
        m = (idv >= base) & (idv < base + RPW)
        lidx = jnp.where(m, idv - base, 0)
        pos = lax.iota(jnp.int32, LL) + i * LL
        plsc.store_scatter(postab, [lidx], pos + 1, mask=m)
        return 0

    lax.fori_loop(0, BB // LL, sb, 0, unroll=8)

    ids0 = plsc.load_gather(ids_v, [zeros])
    for j in range(NCH):
        for kk in range(128 // LL):
            gidx[j, pl.ds(kk * LL, LL)] = zeros
            sidx[j, pl.ds(kk * LL, LL)] = ids0

    def sw(i, cnt):
        idv = ids_v[pl.ds(i * LL, LL)]
        m = (idv >= base) & (idv < base + RPW)
        lidx = jnp.where(m, idv - base, 0)
        pos = lax.iota(jnp.int32, LL) + i * LL
        claimed = plsc.load_gather(postab, [lidx])
        w = m & (claimed == pos + 1)
        mi = w.astype(jnp.int32)
        p = cnt + plsc.cumsum(mi) - 1
        p = jnp.where(w, p, 0)
        plsc.store_scatter(gidx, [p >> 7, p & 127], pos, mask=w)
        plsc.store_scatter(sidx, [p >> 7, p & 127], idv, mask=w)
        return cnt + jnp.sum(mi)

    cnt = lax.fori_loop(0, BB // LL, sw, jnp.int32(0), unroll=4)

    for c in range(NCH):
        @pl.when(c * 128 < cnt)
        def _():
            def build2(i, _):
                k = i >> 3
                l = i & 7
                g = gidx[c, pl.ds(l * LL, LL)]
                s = sidx[c, pl.ds(l * LL, LL)]
                idxg[pl.ds(i * LL, LL)] = g + k * BB
                idxs[k, pl.ds(l * LL, LL)] = s + k * MM
                return 0

            lax.fori_loop(0, DD * 8, build2, 0, unroll=8)

            def gfire(k, _):
                pltpu.async_copy(
                    upd1d.at[idxg.at[pl.ds(k * 128, 128)]],
                    vals.at[pl.ds(k * 128, 128)],
                    gsem,
                )
                return 0

            lax.fori_loop(0, DD, gfire, 0)

            def gdrain(k, _):
                pltpu.make_async_copy(
                    upd1d.at[pl.ds(0, 128)], vals.at[pl.ds(0, 128)],
                    gsem).wait()
                return 0

            lax.fori_loop(0, DD, gdrain, 0)

            def sfire(k, _):
                pltpu.async_copy(
                    vals.at[pl.ds(k * 128, 128)],
                    tref.at[idxs.at[k]],
                    ssem,
                )
                return 0

            lax.fori_loop(0, DD, sfire, 0)

            def sdrain(k, _):
                pltpu.make_async_copy(
                    upd1d.at[pl.ds(0, 128)], vals.at[pl.ds(0, 128)],
                    ssem).wait()
                return 0

            lax.fori_loop(0, DD, sdrain, 0)


def kernel(node_memories, node_ids, messages, W_ih, W_hh, b_ih, b_hh):
    tT = node_memories.T
    hT = _sc_gather(tT.reshape(DD * MM), node_ids)
    updT = _gru_call(
        hT,
        messages.T,
        W_ih.T,
        W_hh.T,
        b_ih.reshape(3 * DD, 1),
        b_hh.reshape(3 * DD, 1),
    )
    table_ref = jax.new_ref(
        _copy_call(tT.reshape(_CROWS, 128)).reshape(DD * MM))
    _sc_scatter(table_ref, node_ids, updT.reshape(DD * BB))
    return jax.freeze(table_ref).reshape(DD, MM).T


# speedup vs baseline: 15.1685x; 15.1685x over previous
import functools

import jax
import jax.numpy as jnp
from jax import lax
from jax.experimental import pallas as pl
from jax.experimental.pallas import tpu as pltpu
from jax.experimental.pallas import tpu_sc as plsc

MM = 1000000
DD = 64
BB = 16384
NC, NS, LL = 2, 16, 16
NW = NC * NS
RPW = MM // NW
BPW = BB // NW
PT = 31264
NCH = 8
TR = 4 * MM

_mesh = plsc.VectorSubcoreMesh(core_axis_name="c", subcore_axis_name="s")
_sc_params = pltpu.CompilerParams(
    use_tc_tiling_on_sc=False, needs_layout_passes=False)


@functools.partial(
    pl.kernel,
    mesh=_mesh,
    out_type=jax.ShapeDtypeStruct((4 * BB, 16), jnp.float32),
    compiler_params=_sc_params,
    scratch_types=[
        pltpu.VMEM((BPW,), jnp.int32),
        pltpu.VMEM((4 * BPW,), jnp.int32),
        pltpu.VMEM((4 * BPW, 16), jnp.float32),
        pltpu.SemaphoreType.DMA,
    ],
)
def _sc_gather(tref, ids, out, idx_v, kbuf, buf, sem):
    wid = lax.axis_index("s") * NC + lax.axis_index("c")
    b0 = wid * BPW
    pltpu.sync_copy(ids.at[pl.ds(b0, BPW)], idx_v)

    def build(i, _):
        p = lax.iota(jnp.int32, LL) + i * LL
        g = plsc.load_gather(idx_v, [p >> 2])
        kbuf[pl.ds(i * LL, LL)] = g * 4 + (p & 3)
        return 0

    lax.fori_loop(0, 4 * BPW // LL, build, 0, unroll=8)

    cps = [
        pltpu.async_copy(
            tref.at[kbuf.at[pl.ds(d * 128, 128)]],
            buf.at[pl.ds(d * 128, 128)],
            sem,
        )
        for d in range(4 * BPW // 128)
    ]
    for c in cps:
        c.wait()
    pltpu.sync_copy(buf, out.at[pl.ds(4 * b0, 4 * BPW)])


def _gru_body(h_ref, m_ref, wit_ref, wht_ref, bi_ref, bh_ref, o_ref):
    h = h_ref[...]
    gi = jnp.dot(m_ref[...], wit_ref[...],
                 preferred_element_type=jnp.float32) + bi_ref[...]
    gh = jnp.dot(h, wht_ref[...],
                 preferred_element_type=jnp.float32) + bh_ref[...]
    r = jax.nn.sigmoid(gi[:, 0:DD] + gh[:, 0:DD])
    z = jax.nn.sigmoid(gi[:, DD:2 * DD] + gh[:, DD:2 * DD])
    n = jnp.tanh(gi[:, 2 * DD:3 * DD] + r * gh[:, 2 * DD:3 * DD])
    o_ref[...] = (1.0 - z) * n + z * h


_GBLK = 2048
_gru_call = pl.pallas_call(
    _gru_body,
    grid=(BB // _GBLK,),
    in_specs=[
        pl.BlockSpec((_GBLK, DD), lambda i: (i, 0)),
        pl.BlockSpec((_GBLK, DD), lambda i: (i, 0)),
        pl.BlockSpec((DD, 3 * DD), lambda i: (0, 0)),
        pl.BlockSpec((DD, 3 * DD), lambda i: (0, 0)),
        pl.BlockSpec((1, 3 * DD), lambda i: (0, 0)),
        pl.BlockSpec((1, 3 * DD), lambda i: (0, 0)),
    ],
    out_specs=pl.BlockSpec((_GBLK, DD), lambda i: (i, 0)),
    out_shape=jax.ShapeDtypeStruct((BB, DD), jnp.float32),
)


@functools.partial(
    pl.kernel,
    mesh=_mesh,
    out_type=(),
    compiler_params=_sc_params,
    scratch_types=[
        pltpu.VMEM((BB,), jnp.int32),
        pltpu.VMEM((PT,), jnp.int32),
        pltpu.VMEM((NCH * 128,), jnp.int32),
        pltpu.VMEM((NCH * 128,), jnp.int32),
        pltpu.VMEM((512,), jnp.int32),
        pltpu.VMEM((4, 128), jnp.int32),
        pltpu.VMEM((512, 16), jnp.float32),
        pltpu.SemaphoreType.DMA,
        pltpu.SemaphoreType.DMA,
    ],
)
def _sc_scatter(tref, ids, upd4, ids_v, postab, gidx, sidx, idxg, idxs,
                vals, gsem, ssem):
    wid = lax.axis_index("s") * NC + lax.axis_index("c")
    base = wid * RPW
    zeros = jnp.zeros((LL,), jnp.int32)

    pltpu.sync_copy(ids, ids_v)

    def za(i, _):
        postab[pl.ds(i * LL, LL)] = zeros
        return 0

    lax.fori_loop(0, PT // LL, za, 0, unroll=8)

    def sb(i, _):
        idv = ids_v[pl.ds(i * LL, LL)]
        m = (idv >= base) & (idv < base + RPW)
        lidx = jnp.where(m, idv - base, 0)
        pos = lax.iota(jnp.int32, LL) + i * LL
        plsc.store_scatter(postab, [lidx], pos + 1, mask=m)
        return 0

    lax.fori_loop(0, BB // LL, sb, 0, unroll=8)

    ids0 = plsc.load_gather(ids_v, [zeros])
    for t in range(NCH * 128 // LL):
        gidx[pl.ds(t * LL, LL)] = zeros
        sidx[pl.ds(t * LL, LL)] = ids0

    def sw(i, cnt):
        idv = ids_v[pl.ds(i * LL, LL)]
        m = (idv >= base) & (idv < base + RPW)
        lidx = jnp.where(m, idv - base, 0)
        pos = lax.iota(jnp.int32, LL) + i * LL
        claimed = plsc.load_gather(postab, [lidx])
        w = m & (claimed == pos + 1)
        mi = w.astype(jnp.int32)
        p = cnt + plsc.cumsum(mi) - 1
        p = jnp.where(w, p, 0)
        plsc.store_scatter(gidx, [p], pos, mask=w)
        plsc.store_scatter(sidx, [p], idv, mask=w)
        return cnt + jnp.sum(mi)

    cnt = lax.fori_loop(0, BB // LL, sw, jnp.int32(0), unroll=4)

    for c in range(NCH):
        @pl.when(c * 128 < cnt)
        def _():
            def build2(i, _):
                p = lax.iota(jnp.int32, LL) + i * LL
                w = c * 128 + (p >> 2)
                g = plsc.load_gather(gidx, [w])
                s = plsc.load_gather(sidx, [w])
                idxg[pl.ds(i * LL, LL)] = g * 4 + (p & 3)
                idxs[i >> 3, pl.ds((i & 7) * LL, LL)] = s * 4 + (p & 3)
                return 0

            lax.fori_loop(0, 32, build2, 0, unroll=8)

            gps = [
                pltpu.async_copy(
                    upd4.at[idxg.at[pl.ds(r * 128, 128)]],
                    vals.at[pl.ds(r * 128, 128)],
                    gsem,
                )
                for r in range(4)
            ]
            for d in gps:
                d.wait()
            sps = [
                pltpu.async_copy(
                    vals.at[pl.ds(r * 128, 128)],
                    tref.at[idxs.at[r]],
                    ssem,
                )
                for r in range(4)
            ]
            for d in sps:
                d.wait()


def kernel(node_memories, node_ids, messages, W_ih, W_hh, b_ih, b_hh):
    t2 = node_memories.reshape(DD * MM // 128, 128)
    table_ref = jax.new_ref(t2.reshape(TR, 16))
    h = _sc_gather(table_ref, node_ids).reshape(BB, DD)
    upd = _gru_call(
        h,
        messages,
        W_ih.T,
        W_hh.T,
        b_ih.reshape(1, 3 * DD),
        b_hh.reshape(1, 3 * DD),
    )
    _sc_scatter(table_ref, node_ids, upd.reshape(4 * BB, 16))
    return jax.freeze(table_ref).reshape(MM, DD)


# speedup vs baseline: 22.7979x; 1.5030x over previous
import functools

import jax
import jax.numpy as jnp
from jax import lax
from jax.experimental import pallas as pl
from jax.experimental.pallas import tpu as pltpu
from jax.experimental.pallas import tpu_sc as plsc

MM = 1000000
DD = 64
BB = 16384
NC, NS, LL = 2, 16, 16
NW = NC * NS
RPW = MM // NW
BPW = BB // NW
PT = 31264
NCH = 8
TR = 8 * MM

_mesh = plsc.VectorSubcoreMesh(core_axis_name="c", subcore_axis_name="s")
_sc_params = pltpu.CompilerParams(
    use_tc_tiling_on_sc=False, needs_layout_passes=False)


@functools.partial(
    pl.kernel,
    mesh=_mesh,
    out_type=jax.ShapeDtypeStruct((4 * BB, 16), jnp.float32),
    compiler_params=_sc_params,
    scratch_types=[
        pltpu.VMEM((BPW,), jnp.int32),
        pltpu.VMEM((4 * BPW,), jnp.int32),
        pltpu.VMEM((4 * BPW, 16), jnp.float32),
        pltpu.SemaphoreType.DMA,
    ],
)
def _sc_gather(tref, ids, out, idx_v, kbuf, buf, sem):
    wid = lax.axis_index("s") * NC + lax.axis_index("c")
    b0 = wid * BPW
    pltpu.sync_copy(ids.at[pl.ds(b0, BPW)], idx_v)

    def build(i, _):
        p = lax.iota(jnp.int32, LL) + i * LL
        g = plsc.load_gather(idx_v, [p >> 2])
        kbuf[pl.ds(i * LL, LL)] = g * 8 + (p & 3)
        return 0

    lax.fori_loop(0, 4 * BPW // LL, build, 0, unroll=8)

    cps = [
        pltpu.async_copy(
            tref.at[kbuf.at[pl.ds(d * 128, 128)]],
            buf.at[pl.ds(d * 128, 128)],
            sem,
        )
        for d in range(4 * BPW // 128)
    ]
    for c in cps:
        c.wait()
    pltpu.sync_copy(buf, out.at[pl.ds(4 * b0, 4 * BPW)])


def _gru_body(h_ref, m_ref, wit_ref, wht_ref, bi_ref, bh_ref, o_ref):
    h = h_ref[...]
    gi = jnp.dot(m_ref[...], wit_ref[...],
                 preferred_element_type=jnp.float32) + bi_ref[...]
    gh = jnp.dot(h, wht_ref[...],
                 preferred_element_type=jnp.float32) + bh_ref[...]
    r = jax.nn.sigmoid(gi[:, 0:DD] + gh[:, 0:DD])
    z = jax.nn.sigmoid(gi[:, DD:2 * DD] + gh[:, DD:2 * DD])
    n = jnp.tanh(gi[:, 2 * DD:3 * DD] + r * gh[:, 2 * DD:3 * DD])
    o_ref[...] = (1.0 - z) * n + z * h


_GBLK = 2048
_gru_call = pl.pallas_call(
    _gru_body,
    grid=(BB // _GBLK,),
    in_specs=[
        pl.BlockSpec((_GBLK, DD), lambda i: (i, 0)),
        pl.BlockSpec((_GBLK, DD), lambda i: (i, 0)),
        pl.BlockSpec((DD, 3 * DD), lambda i: (0, 0)),
        pl.BlockSpec((DD, 3 * DD), lambda i: (0, 0)),
        pl.BlockSpec((1, 3 * DD), lambda i: (0, 0)),
        pl.BlockSpec((1, 3 * DD), lambda i: (0, 0)),
    ],
    out_specs=pl.BlockSpec((_GBLK, DD), lambda i: (i, 0)),
    out_shape=jax.ShapeDtypeStruct((BB, DD), jnp.float32),
)


@functools.partial(
    pl.kernel,
    mesh=_mesh,
    out_type=(),
    compiler_params=_sc_params,
    scratch_types=[
        pltpu.VMEM((BB,), jnp.int32),
        pltpu.VMEM((PT,), jnp.int32),
        pltpu.VMEM((NCH * 128,), jnp.int32),
        pltpu.VMEM((NCH * 128,), jnp.int32),
        pltpu.VMEM((512,), jnp.int32),
        pltpu.VMEM((4, 128), jnp.int32),
        pltpu.VMEM((512, 16), jnp.float32),
        pltpu.SemaphoreType.DMA,
        pltpu.SemaphoreType.DMA,
    ],
)
def _sc_scatter(tref, ids, upd4, ids_v, postab, gidx, sidx, idxg, idxs,
                vals, gsem, ssem):
    wid = lax.axis_index("s") * NC + lax.axis_index("c")
    base = wid * RPW
    zeros = jnp.zeros((LL,), jnp.int32)

    pltpu.sync_copy(ids, ids_v)

    def za(i, _):
        postab[pl.ds(i * LL, LL)] = zeros
        return 0

    lax.fori_loop(0, PT // LL, za, 0, unroll=8)

    def sb(i, _):
        idv = ids_v[pl.ds(i * LL, LL)]
        m = (idv >= base) & (idv < base + RPW)
        lidx = jnp.where(m, idv - base, 0)
        pos = lax.iota(jnp.int32, LL) + i * LL
        plsc.store_scatter(postab, [lidx], pos + 1, mask=m)
        return 0

    lax.fori_loop(0, BB // LL, sb, 0, unroll=8)

    ids0 = plsc.load_gather(ids_v, [zeros])
    for t in range(NCH * 128 // LL):
        gidx[pl.ds(t * LL, LL)] = zeros
        sidx[pl.ds(t * LL, LL)] = ids0

    def sw(i, cnt):
        idv = ids_v[pl.ds(i * LL, LL)]
        m = (idv >= base) & (idv < base + RPW)
        lidx = jnp.where(m, idv - base, 0)
        pos = lax.iota(jnp.int32, LL) + i * LL
        claimed = plsc.load_gather(postab, [lidx])
        w = m & (claimed == pos + 1)
        mi = w.astype(jnp.int32)
        p = cnt + plsc.cumsum(mi) - 1
        p = jnp.where(w, p, 0)
        plsc.store_scatter(gidx, [p], pos, mask=w)
        plsc.store_scatter(sidx, [p], idv, mask=w)
        return cnt + jnp.sum(mi)

    cnt = lax.fori_loop(0, BB // LL, sw, jnp.int32(0), unroll=4)

    for c in range(NCH):
        @pl.when(c * 128 < cnt)
        def _():
            def build2(i, _):
                p = lax.iota(jnp.int32, LL) + i * LL
                w = c * 128 + (p >> 2)
                g = plsc.load_gather(gidx, [w])
                s = plsc.load_gather(sidx, [w])
                idxg[pl.ds(i * LL, LL)] = g * 4 + (p & 3)
                idxs[i >> 3, pl.ds((i & 7) * LL, LL)] = s * 8 + (p & 3)
                return 0

            lax.fori_loop(0, 32, build2, 0, unroll=8)

            gps = [
                pltpu.async_copy(
                    upd4.at[idxg.at[pl.ds(r * 128, 128)]],
                    vals.at[pl.ds(r * 128, 128)],
                    gsem,
                )
                for r in range(4)
            ]
            for d in gps:
                d.wait()
            sps = [
                pltpu.async_copy(
                    vals.at[pl.ds(r * 128, 128)],
                    tref.at[idxs.at[r]],
                    ssem,
                )
                for r in range(4)
            ]
            for d in sps:
                d.wait()


def kernel(node_memories, node_ids, messages, W_ih, W_hh, b_ih, b_hh):
    t2 = jnp.pad(node_memories, ((0, 0), (0, 128 - DD)))
    table_ref = jax.new_ref(t2.reshape(TR, 16))
    h = _sc_gather(table_ref, node_ids).reshape(BB, DD)
    upd = _gru_call(
        h,
        messages,
        W_ih.T,
        W_hh.T,
        b_ih.reshape(1, 3 * DD),
        b_hh.reshape(1, 3 * DD),
    )
    _sc_scatter(table_ref, node_ids, upd.reshape(4 * BB, 16))
    return jax.freeze(table_ref).reshape(MM, 128)[:, :DD]


# speedup vs baseline: 34.3423x; 1.5064x over previous
import functools

import jax
import jax.numpy as jnp
from jax import lax
from jax.experimental import pallas as pl
from jax.experimental.pallas import tpu as pltpu
from jax.experimental.pallas import tpu_sc as plsc

MM = 1000000
DD = 64
BB = 16384
NC, NS, LL = 2, 16, 16
NW = NC * NS
RPW = MM // NW
BPW = BB // NW
PT = 31264
NCH = 8
TR = 8 * MM

_mesh = plsc.VectorSubcoreMesh(core_axis_name="c", subcore_axis_name="s")
_sc_params = pltpu.CompilerParams(
    use_tc_tiling_on_sc=False, needs_layout_passes=False)


@functools.partial(
    pl.kernel,
    mesh=_mesh,
    out_type=jax.ShapeDtypeStruct((4 * BB, 16), jnp.float32),
    compiler_params=_sc_params,
    scratch_types=[
        pltpu.VMEM((BPW,), jnp.int32),
        pltpu.VMEM((4 * BPW,), jnp.int32),
        pltpu.VMEM((4 * BPW, 16), jnp.float32),
        pltpu.SemaphoreType.DMA,
    ],
)
def _sc_gather(tref, ids, out, idx_v, kbuf, buf, sem):
    wid = lax.axis_index("s") * NC + lax.axis_index("c")
    b0 = wid * BPW
    pltpu.sync_copy(ids.at[pl.ds(b0, BPW)], idx_v)

    def build(i, _):
        p = lax.iota(jnp.int32, LL) + i * LL
        g = plsc.load_gather(idx_v, [p >> 2])
        kbuf[pl.ds(i * LL, LL)] = g * 8 + (p & 3)
        return 0

    lax.fori_loop(0, 4 * BPW // LL, build, 0, unroll=8)

    cps = [
        pltpu.async_copy(
            tref.at[kbuf.at[pl.ds(d * 128, 128)]],
            buf.at[pl.ds(d * 128, 128)],
            sem,
        )
        for d in range(4 * BPW // 128)
    ]
    for c in cps:
        c.wait()
    pltpu.sync_copy(buf, out.at[pl.ds(4 * b0, 4 * BPW)])


def _gru_body(h_ref, m_ref, wit_ref, wht_ref, bi_ref, bh_ref, o_ref):
    h = h_ref[...]
    gi = jnp.dot(m_ref[...], wit_ref[...],
                 preferred_element_type=jnp.float32) + bi_ref[...]
    gh = jnp.dot(h, wht_ref[...],
                 preferred_element_type=jnp.float32) + bh_ref[...]
    r = jax.nn.sigmoid(gi[:, 0:DD] + gh[:, 0:DD])
    z = jax.nn.sigmoid(gi[:, DD:2 * DD] + gh[:, DD:2 * DD])
    n = jnp.tanh(gi[:, 2 * DD:3 * DD] + r * gh[:, 2 * DD:3 * DD])
    o_ref[...] = (1.0 - z) * n + z * h


_GBLK = 2048
_gru_call = pl.pallas_call(
    _gru_body,
    grid=(BB // _GBLK,),
    in_specs=[
        pl.BlockSpec((_GBLK, DD), lambda i: (i, 0)),
        pl.BlockSpec((_GBLK, DD), lambda i: (i, 0)),
        pl.BlockSpec((DD, 3 * DD), lambda i: (0, 0)),
        pl.BlockSpec((DD, 3 * DD), lambda i: (0, 0)),
        pl.BlockSpec((1, 3 * DD), lambda i: (0, 0)),
        pl.BlockSpec((1, 3 * DD), lambda i: (0, 0)),
    ],
    out_specs=pl.BlockSpec((_GBLK, DD), lambda i: (i, 0)),
    out_shape=jax.ShapeDtypeStruct((BB, DD), jnp.float32),
)


@functools.partial(
    pl.kernel,
    mesh=_mesh,
    out_type=(),
    compiler_params=_sc_params,
    scratch_types=[
        pltpu.VMEM((BB,), jnp.int32),
        pltpu.VMEM((PT,), jnp.int32),
        pltpu.VMEM((NCH * 128,), jnp.int32),
        pltpu.VMEM((NCH * 128,), jnp.int32),
        pltpu.VMEM((512,), jnp.int32),
        pltpu.VMEM((4, 128), jnp.int32),
        pltpu.VMEM((512, 16), jnp.float32),
        pltpu.SemaphoreType.DMA,
        pltpu.SemaphoreType.DMA,
    ],
)
def _sc_scatter(tref, ids, upd4, ids_v, postab, gidx, sidx, idxg, idxs,
                vals, gsem, ssem):
    wid = lax.axis_index("s") * NC + lax.axis_index("c")
    base = wid * RPW
    zeros = jnp.zeros((LL,), jnp.int32)

    pltpu.sync_copy(ids, ids_v)

    def za(i, _):
        postab[pl.ds(i * LL, LL)] = zeros
        return 0

    lax.fori_loop(0, PT // LL, za, 0, unroll=8)

    def sb(i, _):
        idv = ids_v[pl.ds(i * LL, LL)]
        m = (idv >= base) & (idv < base + RPW)
        lidx = jnp.where(m, idv - base, 0)
        pos = lax.iota(jnp.int32, LL) + i * LL
        plsc.store_scatter(postab, [lidx], pos + 1, mask=m)
        return 0

    lax.fori_loop(0, BB // LL, sb, 0, unroll=8)

    ids0 = plsc.load_gather(ids_v, [zeros])
    for t in range(NCH * 128 // LL):
        gidx[pl.ds(t * LL, LL)] = zeros
        sidx[pl.ds(t * LL, LL)] = ids0

    def sw(i, cnt):
        idv = ids_v[pl.ds(i * LL, LL)]
        m = (idv >= base) & (idv < base + RPW)
        lidx = jnp.where(m, idv - base, 0)
        pos = lax.iota(jnp.int32, LL) + i * LL
        claimed = plsc.load_gather(postab, [lidx])
        w = m & (claimed == pos + 1)
        mi = w.astype(jnp.int32)
        p = cnt + plsc.cumsum(mi) - 1
        p = jnp.where(w, p, 0)
        plsc.store_scatter(gidx, [p], pos, mask=w)
        plsc.store_scatter(sidx, [p], idv, mask=w)
        return cnt + jnp.sum(mi)

    cnt = lax.fori_loop(0, BB // LL, sw, jnp.int32(0), unroll=4)

    for c in range(NCH):
        @pl.when(c * 128 < cnt)
        def _():
            def build2(i, _):
                p = lax.iota(jnp.int32, LL) + i * LL
                w = c * 128 + (p >> 2)
                g = plsc.load_gather(gidx, [w])
                s = plsc.load_gather(sidx, [w])
                idxg[pl.ds(i * LL, LL)] = g * 4 + (p & 3)
                idxs[i >> 3, pl.ds((i & 7) * LL, LL)] = s * 8 + (p & 3)
                return 0

            lax.fori_loop(0, 32, build2, 0, unroll=8)

            gps = [
                pltpu.async_copy(
                    upd4.at[idxg.at[pl.ds(r * 128, 128)]],
                    vals.at[pl.ds(r * 128, 128)],
                    gsem,
                )
                for r in range(4)
            ]
            for d in gps:
                d.wait()
            sps = [
                pltpu.async_copy(
                    vals.at[pl.ds(r * 128, 128)],
                    tref.at[idxs.at[r]],
                    ssem,
                )
                for r in range(4)
            ]
            for d in sps:
                d.wait()


def kernel(node_memories, node_ids, messages, W_ih, W_hh, b_ih, b_hh):
    eyep = jnp.eye(DD, 128, dtype=jnp.float32)
    t2 = jnp.dot(node_memories, eyep, preferred_element_type=jnp.float32)
    table_ref = jax.new_ref(t2.reshape(TR, 16))
    h = _sc_gather(table_ref, node_ids).reshape(BB, DD)
    upd = _gru_call(
        h,
        messages,
        W_ih.T,
        W_hh.T,
        b_ih.reshape(1, 3 * DD),
        b_hh.reshape(1, 3 * DD),
    )
    _sc_scatter(table_ref, node_ids, upd.reshape(4 * BB, 16))
    return jax.freeze(table_ref).reshape(MM, 128)[:, :DD]
